# Initial kernel scaffold; baseline (speedup 1.0000x reference)
import jax
import jax.numpy as jnp
from jax.experimental import pallas as pl


def kernel(x, table):
    raise NotImplementedError("write your pallas kernel here")


# speedup vs baseline: 2.3786x; 2.3786x over previous
import functools
import math

import jax
import jax.numpy as jnp
import numpy as np
from jax import lax
from jax.experimental import pallas as pl
from jax.experimental.pallas import tpu as pltpu
from jax.experimental.pallas import tpu_sc as plsc

VOCAB = 100000
EMB = 126
D = 128
B = 1024
L = 200
NC = 2
NS = 16
NW = NC * NS
BPW = B // NW
SCALE = math.sqrt(float(D))
GCH = 100
NGR = (L + 15) // 16


def _enc_const() -> np.ndarray:
    position = np.arange(L, dtype=np.float32)[:, None]
    div_term = np.exp(np.arange(0, D, 2, dtype=np.float32) * (-math.log(10000.0) / D))
    enc = np.zeros((L, D), dtype=np.float32)
    enc[:, 0::2] = np.sin(position * div_term)
    enc[:, 1::2] = np.cos(position * div_term)
    return enc


_MESH = plsc.VectorSubcoreMesh(core_axis_name="c", subcore_axis_name="s")


@functools.partial(
    pl.kernel,
    mesh=_MESH,
    out_type=jax.ShapeDtypeStruct((B, L, D), jnp.float32),
    scratch_types=[
        pltpu.VMEM((BPW, 2, GCH), jnp.int32),
        pltpu.VMEM((L, D), jnp.float32),
        pltpu.VMEM((L, D), jnp.float32),
        pltpu.VMEM((L, 16), jnp.float32),
        pltpu.SemaphoreType.DMA,
    ],
)
def _sc_embed(tab_hbm, idx_hbm, thick_hbm, enc_hbm, out_hbm,
              idx_v, enc_v, emb_v, thick_v, sem):
    wid = lax.axis_index("s") * NC + lax.axis_index("c")
    pltpu.sync_copy(idx_hbm.at[wid], idx_v)
    pltpu.sync_copy(enc_hbm, enc_v)

    def batch_body(bb, carry):
        b = wid * BPW + bb
        pltpu.sync_copy(thick_hbm.at[wid, bb], thick_v)
        cp0 = pltpu.async_copy(tab_hbm.at[idx_v.at[bb, 0]],
                               emb_v.at[pl.ds(0, GCH)], sem)
        cp1 = pltpu.async_copy(tab_hbm.at[idx_v.at[bb, 1]],
                               emb_v.at[pl.ds(GCH, GCH)], sem)
        cp0.wait()
        cp1.wait()

        def row_body(r, c):
            for k in range(D // 16 - 1):
                sl = pl.ds(k * 16, 16)
                emb_v[r, sl] = emb_v[r, sl] * SCALE + enc_v[r, sl]
            sl = pl.ds(D - 16, 16)
            emb_v[r, sl] = (emb_v[r, sl] + thick_v[r, :]) * SCALE + enc_v[r, sl]
            return c

        lax.fori_loop(0, L, row_body, 0)

        pltpu.sync_copy(emb_v, out_hbm.at[b])
        return carry

    lax.fori_loop(0, BPW, batch_body, 0)


def kernel(x, table):
    idx = x[:, :, 0].astype(jnp.int32).reshape(NW, BPW, 2, GCH)
    thick16 = jnp.pad(x[:, :, 1:], ((0, 0), (0, 0), (14, 0)))
    thick16 = thick16.reshape(NW, BPW, L, 16)
    tab = jnp.pad(table, ((0, 0), (0, D - EMB)))
    enc = jnp.asarray(_enc_const())
    return _sc_embed(tab, idx, thick16, enc)
